# baseline (speedup 1.0000x reference)
import jax
import jax.numpy as jnp
from jax import lax
from jax.experimental import pallas as pl
from jax.experimental.pallas import tpu as pltpu
from jax.experimental.pallas import tpu_sc as plsc

_P_DIM = 64
_T_DIM = 32
_NB = 10
_COND = 256
_B = 16384
_L = 16
_NPE = 51
_NTI = 21
_NTB = _NTI * _NB
_TROWS = _NPE + _NTB + 1

_NC = 2
_NS = 16
_NW = _NC * _NS
_BPW = _B // _NW
_CH = 64
_GPC = _CH // _L
_NGRP = _BPW // _L


def _tables_body(pe_ref, te_ref, w_ref, b_ref, p_ref, t_ref, bi_ref,
                 tab_out, wm_out, pk_out):
    w = w_ref[...]
    dn = (((1,), (1,)), ((), ()))
    ptab = (
        lax.dot_general(pe_ref[...], w[:, :_P_DIM], dn,
                        preferred_element_type=jnp.float32)
        + b_ref[...]
    )
    t_tab = lax.dot_general(te_ref[...], w[:, _P_DIM:_P_DIM + _T_DIM], dn,
                            preferred_element_type=jnp.float32)
    i10 = (lax.broadcasted_iota(jnp.int32, (_NB, _NB), 0)
           == lax.broadcasted_iota(jnp.int32, (_NB, _NB), 1)).astype(jnp.float32)
    b_tab = lax.dot_general(i10, w[:, _P_DIM + _T_DIM:_P_DIM + _T_DIM + _NB],
                            dn, preferred_element_type=jnp.float32)
    r21 = lax.broadcasted_iota(jnp.int32, (_NTB, _NTI), 0) // _NB
    o1 = (r21 == lax.broadcasted_iota(jnp.int32, (_NTB, _NTI), 1)).astype(jnp.float32)
    r10 = lax.broadcasted_iota(jnp.int32, (_NTB, _NB), 0) % _NB
    o2 = (r10 == lax.broadcasted_iota(jnp.int32, (_NTB, _NB), 1)).astype(jnp.float32)
    mm = (((1,), (0,)), ((), ()))
    tbtab = (
        lax.dot_general(o1, t_tab, mm, preferred_element_type=jnp.float32)
        + lax.dot_general(o2, b_tab, mm, preferred_element_type=jnp.float32)
    )
    e106 = (lax.broadcasted_iota(jnp.int32, (1, _P_DIM + _T_DIM + _NB + 1), 1)
            == (_P_DIM + _T_DIM + _NB)).astype(jnp.float32)
    wm = lax.dot_general(e106, w, dn, preferred_element_type=jnp.float32)
    tab_out[...] = jnp.concatenate([ptab, tbtab, wm], axis=0)
    wm_out[...] = wm
    pk_out[...] = (p_ref[...] + t_ref[...] * (_NB * 64) + bi_ref[...] * 64
                   + _NPE * 64)


_tables = pl.pallas_call(
    _tables_body,
    out_shape=(
        jax.ShapeDtypeStruct((_TROWS, _COND), jnp.float32),
        jax.ShapeDtypeStruct((1, _COND), jnp.float32),
        jax.ShapeDtypeStruct((_B,), jnp.int32),
    ),
)


def _sc_body(m_hbm, pk_hbm, tab_hbm, wm_hbm, out_hbm,
             pk_v, mloy_v, tab_v, wm_v, obuf, isem, osem0, osem1):
    cid = lax.axis_index("c")
    sid = lax.axis_index("s")
    wid = sid * _NC + cid
    base = wid * _BPW
    ctab = pltpu.make_async_copy(tab_hbm, tab_v, isem)
    cwm = pltpu.make_async_copy(wm_hbm, wm_v, isem)
    cpk = pltpu.make_async_copy(pk_hbm.at[pl.ds(base, _BPW)], pk_v, isem)
    cm = pltpu.make_async_copy(m_hbm.at[pl.ds(base, _BPW)], mloy_v, isem)
    ctab.start()
    cwm.start()
    cpk.start()
    cm.start()
    ctab.wait()
    cwm.wait()
    cpk.wait()
    cm.wait()

    wm_regs = [wm_v[0, pl.ds(c * _L, _L)] for c in range(_COND // _L)]

    def _drain(sem):
        pltpu.make_async_copy(out_hbm.at[pl.ds(0, _CH), :],
                              obuf.at[pl.ds(0, _CH), :], sem).wait()

    def group_body(g, _):
        @pl.when(jnp.logical_and(g % (2 * _GPC) == 0, g >= 2 * _GPC))
        def _():
            _drain(osem0)

        @pl.when(jnp.logical_and(g % (2 * _GPC) == _GPC, g >= 2 * _GPC))
        def _():
            _drain(osem1)

        parity = (g // _GPC) % 2
        brow0 = parity * _CH + (g % _GPC) * _L
        mvec = mloy_v[pl.ds(g * _L, _L)]
        pkvec = pk_v[pl.ds(g * _L, _L)]
        pks = [pkvec[l] for l in range(_L)]
        for l in range(_L):
            pa = lax.rem(pks[l], 64) * _COND
            ta = lax.div(pks[l], 64) * _COND
            ms = jnp.full((_L,), mvec[l], jnp.float32)
            br = brow0 + l
            vps = [tab_v[pl.ds(pa + c * _L, _L)]
                   for c in range(_COND // _L)]
            vts = [tab_v[pl.ds(ta + c * _L, _L)]
                   for c in range(_COND // _L)]
            for c in range(_COND // _L):
                obuf[br, pl.ds(c * _L, _L)] = (
                    vps[c] + vts[c] + ms * wm_regs[c])

        @pl.when(g % (2 * _GPC) == _GPC - 1)
        def _():
            pltpu.async_copy(
                obuf.at[pl.ds(0, _CH), :],
                out_hbm.at[pl.ds(base + (g // _GPC) * _CH, _CH), :],
                osem0)

        @pl.when(g % (2 * _GPC) == 2 * _GPC - 1)
        def _():
            pltpu.async_copy(
                obuf.at[pl.ds(_CH, _CH), :],
                out_hbm.at[pl.ds(base + (g // _GPC) * _CH, _CH), :],
                osem1)

        return 0

    lax.fori_loop(0, _NGRP, group_body, 0)
    _drain(osem0)
    _drain(osem1)


def _make_sc_gather():
  return pl.kernel(
    _sc_body,
    out_type=jax.ShapeDtypeStruct((_B, _COND), jnp.float32),
    mesh=plsc.VectorSubcoreMesh(core_axis_name="c", subcore_axis_name="s",
                                num_cores=_NC, num_subcores=_NS),
    scratch_types=[
        pltpu.VMEM((_BPW,), jnp.int32),
        pltpu.VMEM((_BPW,), jnp.float32),
        pltpu.VMEM((_TROWS * _COND,), jnp.float32),
        pltpu.VMEM((1, _COND), jnp.float32),
        pltpu.VMEM((2 * _CH, _COND), jnp.float32),
        pltpu.SemaphoreType.DMA,
        pltpu.SemaphoreType.DMA,
        pltpu.SemaphoreType.DMA,
    ],
  )


_sc_gather_cache = []


def kernel(p_idx, t_idx, b_idx, mLOY, perturb_embedding, tissue_embedding, W, b):
    if not _sc_gather_cache:
        _sc_gather_cache.append(_make_sc_gather())
    sc_gather = _sc_gather_cache[0]
    tab, wm, pk = _tables(perturb_embedding, tissue_embedding, W,
                          b.reshape(1, _COND), p_idx.astype(jnp.int32),
                          t_idx.astype(jnp.int32), b_idx.astype(jnp.int32))
    return sc_gather(mLOY.astype(jnp.float32), pk, tab.reshape(-1), wm)


# speedup vs baseline: 1.0443x; 1.0443x over previous
import jax
import jax.numpy as jnp
from jax import lax
from jax.experimental import pallas as pl
from jax.experimental.pallas import tpu as pltpu
from jax.experimental.pallas import tpu_sc as plsc

_P_DIM = 64
_T_DIM = 32
_NB = 10
_COND = 256
_B = 16384
_L = 16
_NPE = 51
_NTI = 21
_NTB = _NTI * _NB
_TROWS = _NPE + _NTB + 1

_NC = 2
_NS = 16
_NW = _NC * _NS
_BPW = _B // _NW
_CH = 64
_GPC = _CH // _L
_NGRP = _BPW // _L


def _tables_body(pe_ref, te_ref, w_ref, b_ref, p_ref, t_ref, bi_ref,
                 tab_out, wm_out, pk_out):
    w = w_ref[...]
    dn = (((1,), (1,)), ((), ()))
    ptab = (
        lax.dot_general(pe_ref[...], w[:, :_P_DIM], dn,
                        preferred_element_type=jnp.float32)
        + b_ref[...]
    )
    t_tab = lax.dot_general(te_ref[...], w[:, _P_DIM:_P_DIM + _T_DIM], dn,
                            preferred_element_type=jnp.float32)
    i10 = (lax.broadcasted_iota(jnp.int32, (_NB, _NB), 0)
           == lax.broadcasted_iota(jnp.int32, (_NB, _NB), 1)).astype(jnp.float32)
    b_tab = lax.dot_general(i10, w[:, _P_DIM + _T_DIM:_P_DIM + _T_DIM + _NB],
                            dn, preferred_element_type=jnp.float32)
    r21 = lax.broadcasted_iota(jnp.int32, (_NTB, _NTI), 0) // _NB
    o1 = (r21 == lax.broadcasted_iota(jnp.int32, (_NTB, _NTI), 1)).astype(jnp.float32)
    r10 = lax.broadcasted_iota(jnp.int32, (_NTB, _NB), 0) % _NB
    o2 = (r10 == lax.broadcasted_iota(jnp.int32, (_NTB, _NB), 1)).astype(jnp.float32)
    mm = (((1,), (0,)), ((), ()))
    tbtab = (
        lax.dot_general(o1, t_tab, mm, preferred_element_type=jnp.float32)
        + lax.dot_general(o2, b_tab, mm, preferred_element_type=jnp.float32)
    )
    e106 = (lax.broadcasted_iota(jnp.int32, (1, _P_DIM + _T_DIM + _NB + 1), 1)
            == (_P_DIM + _T_DIM + _NB)).astype(jnp.float32)
    wm = lax.dot_general(e106, w, dn, preferred_element_type=jnp.float32)
    tab_out[...] = jnp.concatenate([ptab, tbtab, wm], axis=0)
    wm_out[...] = wm
    pk_out[...] = (p_ref[...] + t_ref[...] * (_NB * 64) + bi_ref[...] * 64
                   + _NPE * 64)


_tables = pl.pallas_call(
    _tables_body,
    out_shape=(
        jax.ShapeDtypeStruct((_TROWS, _COND), jnp.float32),
        jax.ShapeDtypeStruct((1, _COND), jnp.float32),
        jax.ShapeDtypeStruct((_B,), jnp.int32),
    ),
)


def _sc_body(m_hbm, pk_hbm, tab_hbm, wm_hbm, out_hbm,
             pk_v, mloy_v, tab_v, wm_v, obuf, isem, osem0, osem1):
    cid = lax.axis_index("c")
    sid = lax.axis_index("s")
    wid = sid * _NC + cid
    base = wid * _BPW
    ctab = pltpu.make_async_copy(tab_hbm, tab_v, isem)
    cwm = pltpu.make_async_copy(wm_hbm, wm_v, isem)
    cpk = pltpu.make_async_copy(pk_hbm.at[pl.ds(base, _BPW)], pk_v, isem)
    cm = pltpu.make_async_copy(m_hbm.at[pl.ds(base, _BPW)], mloy_v, isem)
    ctab.start()
    cwm.start()
    cpk.start()
    cm.start()
    ctab.wait()
    cwm.wait()
    cpk.wait()
    cm.wait()

    wm_regs = [wm_v[0, pl.ds(c * _L, _L)] for c in range(_COND // _L)]

    def _drain(sem):
        pltpu.make_async_copy(out_hbm.at[pl.ds(0, _CH), :],
                              obuf.at[pl.ds(0, _CH), :], sem).wait()

    def group_body(g, _):
        @pl.when(jnp.logical_and(g % (2 * _GPC) == 0, g >= 2 * _GPC))
        def _():
            _drain(osem0)

        @pl.when(jnp.logical_and(g % (2 * _GPC) == _GPC, g >= 2 * _GPC))
        def _():
            _drain(osem1)

        parity = (g // _GPC) % 2
        brow0 = parity * _CH + (g % _GPC) * _L
        mvec = mloy_v[pl.ds(g * _L, _L)]
        pkvec = pk_v[pl.ds(g * _L, _L)]
        pks = [pkvec[l] for l in range(_L)]
        for l in range(_L):
            pa = lax.rem(pks[l], 64)
            ta = lax.div(pks[l], 64)
            ms = jnp.full((_L,), mvec[l], jnp.float32)
            br = brow0 + l
            vps = [tab_v[pa, pl.ds(c * _L, _L)]
                   for c in range(_COND // _L)]
            vts = [tab_v[ta, pl.ds(c * _L, _L)]
                   for c in range(_COND // _L)]
            for c in range(_COND // _L):
                obuf[br, pl.ds(c * _L, _L)] = (
                    vps[c] + vts[c] + ms * wm_regs[c])

        @pl.when(g % (2 * _GPC) == _GPC - 1)
        def _():
            pltpu.async_copy(
                obuf.at[pl.ds(0, _CH), :],
                out_hbm.at[pl.ds(base + (g // _GPC) * _CH, _CH), :],
                osem0)

        @pl.when(g % (2 * _GPC) == 2 * _GPC - 1)
        def _():
            pltpu.async_copy(
                obuf.at[pl.ds(_CH, _CH), :],
                out_hbm.at[pl.ds(base + (g // _GPC) * _CH, _CH), :],
                osem1)

        return 0

    lax.fori_loop(0, _NGRP, group_body, 0)
    _drain(osem0)
    _drain(osem1)


def _make_sc_gather():
  return pl.kernel(
    _sc_body,
    out_type=jax.ShapeDtypeStruct((_B, _COND), jnp.float32),
    mesh=plsc.VectorSubcoreMesh(core_axis_name="c", subcore_axis_name="s",
                                num_cores=_NC, num_subcores=_NS),
    scratch_types=[
        pltpu.VMEM((_BPW,), jnp.int32),
        pltpu.VMEM((_BPW,), jnp.float32),
        pltpu.VMEM((_TROWS, _COND), jnp.float32),
        pltpu.VMEM((1, _COND), jnp.float32),
        pltpu.VMEM((2 * _CH, _COND), jnp.float32),
        pltpu.SemaphoreType.DMA,
        pltpu.SemaphoreType.DMA,
        pltpu.SemaphoreType.DMA,
    ],
  )


_sc_gather_cache = []


def kernel(p_idx, t_idx, b_idx, mLOY, perturb_embedding, tissue_embedding, W, b):
    if not _sc_gather_cache:
        _sc_gather_cache.append(_make_sc_gather())
    sc_gather = _sc_gather_cache[0]
    tab, wm, pk = _tables(perturb_embedding, tissue_embedding, W,
                          b.reshape(1, _COND), p_idx.astype(jnp.int32),
                          t_idx.astype(jnp.int32), b_idx.astype(jnp.int32))
    return sc_gather(mLOY.astype(jnp.float32), pk, tab, wm)


# speedup vs baseline: 1.0547x; 1.0099x over previous
import jax
import jax.numpy as jnp
from jax import lax
from jax.experimental import pallas as pl
from jax.experimental.pallas import tpu as pltpu
from jax.experimental.pallas import tpu_sc as plsc

_P_DIM = 64
_T_DIM = 32
_NB = 10
_COND = 256
_B = 16384
_L = 16
_NPE = 51
_NTI = 21
_NTB = _NTI * _NB
_TROWS = _NPE + _NTB + 1

_NC = 2
_NS = 16
_NW = _NC * _NS
_BPW = _B // _NW
_CH = 64
_GPC = _CH // _L
_NGRP = _BPW // _L


def _tables_body(pe_ref, te_ref, w_ref, b_ref, p_ref, t_ref, bi_ref,
                 tab_out, wm_out, pk_out):
    w = w_ref[...]
    dn = (((1,), (1,)), ((), ()))
    ptab = (
        lax.dot_general(pe_ref[...], w[:, :_P_DIM], dn,
                        preferred_element_type=jnp.float32)
        + b_ref[...]
    )
    t_tab = lax.dot_general(te_ref[...], w[:, _P_DIM:_P_DIM + _T_DIM], dn,
                            preferred_element_type=jnp.float32)
    i10 = (lax.broadcasted_iota(jnp.int32, (_NB, _NB), 0)
           == lax.broadcasted_iota(jnp.int32, (_NB, _NB), 1)).astype(jnp.float32)
    b_tab = lax.dot_general(i10, w[:, _P_DIM + _T_DIM:_P_DIM + _T_DIM + _NB],
                            dn, preferred_element_type=jnp.float32)
    r21 = lax.broadcasted_iota(jnp.int32, (_NTB, _NTI), 0) // _NB
    o1 = (r21 == lax.broadcasted_iota(jnp.int32, (_NTB, _NTI), 1)).astype(jnp.float32)
    r10 = lax.broadcasted_iota(jnp.int32, (_NTB, _NB), 0) % _NB
    o2 = (r10 == lax.broadcasted_iota(jnp.int32, (_NTB, _NB), 1)).astype(jnp.float32)
    mm = (((1,), (0,)), ((), ()))
    tbtab = (
        lax.dot_general(o1, t_tab, mm, preferred_element_type=jnp.float32)
        + lax.dot_general(o2, b_tab, mm, preferred_element_type=jnp.float32)
    )
    e106 = (lax.broadcasted_iota(jnp.int32, (1, _P_DIM + _T_DIM + _NB + 1), 1)
            == (_P_DIM + _T_DIM + _NB)).astype(jnp.float32)
    wm = lax.dot_general(e106, w, dn, preferred_element_type=jnp.float32)
    tab_out[...] = jnp.concatenate([ptab, tbtab, wm], axis=0)
    wm_out[...] = wm
    pk_out[...] = (p_ref[...] + t_ref[...] * (_NB * 64) + bi_ref[...] * 64
                   + _NPE * 64)


_tables = pl.pallas_call(
    _tables_body,
    out_shape=(
        jax.ShapeDtypeStruct((_TROWS, _COND), jnp.float32),
        jax.ShapeDtypeStruct((1, _COND), jnp.float32),
        jax.ShapeDtypeStruct((_B,), jnp.int32),
    ),
)


def _sc_body(m_hbm, pk_hbm, tab_hbm, wm_hbm, out_hbm,
             pk_v, mloy_v, tab_v, wm_v, obuf, isem, osem0, osem1):
    cid = lax.axis_index("c")
    sid = lax.axis_index("s")
    wid = sid * _NC + cid
    base = wid * _BPW
    ctab = pltpu.make_async_copy(tab_hbm, tab_v, isem)
    cwm = pltpu.make_async_copy(wm_hbm, wm_v, isem)
    cpk = pltpu.make_async_copy(pk_hbm.at[pl.ds(base, _BPW)], pk_v, isem)
    cm = pltpu.make_async_copy(m_hbm.at[pl.ds(base, _BPW)], mloy_v, isem)
    ctab.start()
    cwm.start()
    cpk.start()
    cm.start()
    ctab.wait()
    cwm.wait()
    cpk.wait()
    cm.wait()

    wm_regs = [wm_v[0, pl.ds(c * _L, _L)] for c in range(_COND // _L)]

    def _drain(sem):
        pltpu.make_async_copy(out_hbm.at[pl.ds(0, _CH), :],
                              obuf.at[pl.ds(0, _CH), :], sem).wait()

    def group_body(g, _):
        @pl.when(jnp.logical_and(g % (2 * _GPC) == 0, g >= 2 * _GPC))
        def _():
            _drain(osem0)

        @pl.when(jnp.logical_and(g % (2 * _GPC) == _GPC, g >= 2 * _GPC))
        def _():
            _drain(osem1)

        parity = (g // _GPC) % 2
        brow0 = parity * _CH + (g % _GPC) * _L
        mvec = mloy_v[pl.ds(g * _L, _L)]
        pkvec = pk_v[pl.ds(g * _L, _L)]
        pks = [pkvec[l] for l in range(_L)]

        def _flush(st):
            pbr, ph, pvps, pvts, pms = st
            for c in range(8):
                cc = ph * 8 + c
                obuf[pbr, pl.ds(cc * _L, _L)] = (
                    pvps[c] + pvts[c] + pms * wm_regs[cc])

        pend = None
        for l in range(_L):
            pa = lax.rem(pks[l], 64)
            ta = lax.div(pks[l], 64)
            ms = jnp.full((_L,), mvec[l], jnp.float32)
            br = brow0 + l
            for h in range(2):
                vps = [tab_v[pa, pl.ds((h * 8 + c) * _L, _L)]
                       for c in range(8)]
                vts = [tab_v[ta, pl.ds((h * 8 + c) * _L, _L)]
                       for c in range(8)]
                if pend is not None:
                    _flush(pend)
                pend = (br, h, vps, vts, ms)
        _flush(pend)

        @pl.when(g % (2 * _GPC) == _GPC - 1)
        def _():
            pltpu.async_copy(
                obuf.at[pl.ds(0, _CH), :],
                out_hbm.at[pl.ds(base + (g // _GPC) * _CH, _CH), :],
                osem0)

        @pl.when(g % (2 * _GPC) == 2 * _GPC - 1)
        def _():
            pltpu.async_copy(
                obuf.at[pl.ds(_CH, _CH), :],
                out_hbm.at[pl.ds(base + (g // _GPC) * _CH, _CH), :],
                osem1)

        return 0

    lax.fori_loop(0, _NGRP, group_body, 0)
    _drain(osem0)
    _drain(osem1)


def _make_sc_gather():
  return pl.kernel(
    _sc_body,
    out_type=jax.ShapeDtypeStruct((_B, _COND), jnp.float32),
    mesh=plsc.VectorSubcoreMesh(core_axis_name="c", subcore_axis_name="s",
                                num_cores=_NC, num_subcores=_NS),
    scratch_types=[
        pltpu.VMEM((_BPW,), jnp.int32),
        pltpu.VMEM((_BPW,), jnp.float32),
        pltpu.VMEM((_TROWS, _COND), jnp.float32),
        pltpu.VMEM((1, _COND), jnp.float32),
        pltpu.VMEM((2 * _CH, _COND), jnp.float32),
        pltpu.SemaphoreType.DMA,
        pltpu.SemaphoreType.DMA,
        pltpu.SemaphoreType.DMA,
    ],
  )


_sc_gather_cache = []


def kernel(p_idx, t_idx, b_idx, mLOY, perturb_embedding, tissue_embedding, W, b):
    if not _sc_gather_cache:
        _sc_gather_cache.append(_make_sc_gather())
    sc_gather = _sc_gather_cache[0]
    tab, wm, pk = _tables(perturb_embedding, tissue_embedding, W,
                          b.reshape(1, _COND), p_idx.astype(jnp.int32),
                          t_idx.astype(jnp.int32), b_idx.astype(jnp.int32))
    return sc_gather(mLOY.astype(jnp.float32), pk, tab, wm)
